# baseline (speedup 1.0000x reference)
import functools

import jax
import jax.numpy as jnp
from jax import lax
from jax.experimental import pallas as pl
from jax.experimental.pallas import tpu as pltpu
from jax.experimental.pallas import tpu_sc as plsc

F = 64
B = 16384
SEG = 128
N_ROWS = 1000000
R = 1024
NB = HALF_BLOCKS = 488
HALF = NB * R
TAIL = 2 * HALF
OUT_ROWS = HALF + (N_ROWS - TAIL)


def _fuse_tables_tc(ut, vt):
    def body(u_top, u_bot, v_top, v_bot, u_out, v_out):
        u_out[...] = jnp.concatenate([u_top[...].T, u_bot[...].T], axis=1)
        v_out[...] = jnp.concatenate([v_top[...].T, v_bot[...].T], axis=1)

    def top_map(i):
        return (0, jnp.where(i < NB, i, 2 * NB))

    def bot_map(i):
        return (0, jnp.where(i < NB, i + NB, 2 * NB))

    top_spec = pl.BlockSpec((F, R), top_map)
    bot_spec = pl.BlockSpec((F, R), bot_map)
    out_spec = pl.BlockSpec((R, 2 * F), lambda i: (i, 0))
    out_shape = jax.ShapeDtypeStruct((OUT_ROWS, 2 * F), jnp.float32)
    return pl.pallas_call(
        body,
        grid=(NB + 1,),
        in_specs=[top_spec, bot_spec, top_spec, bot_spec],
        out_specs=[out_spec, out_spec],
        out_shape=[out_shape, out_shape],
    )(ut, ut, vt, vt)


def _gmf_sc(x1r, x2r, u_fused, v_fused, wb):
    info = plsc.get_sparse_core_info()
    nw = info.num_cores * info.num_subcores
    b_per_w = B // nw
    n_seg = b_per_w // SEG

    mesh = plsc.VectorSubcoreMesh(core_axis_name="c", subcore_axis_name="s")

    @functools.partial(
        pl.kernel,
        mesh=mesh,
        out_type=jax.ShapeDtypeStruct((B,), jnp.float32),
        scratch_types=[
            pltpu.VMEM((n_seg, SEG), jnp.int32),
            pltpu.VMEM((n_seg, SEG), jnp.int32),
            pltpu.VMEM((n_seg, SEG), jnp.int32),
            pltpu.VMEM((n_seg, SEG), jnp.int32),
            pltpu.VMEM((b_per_w,), jnp.int32),
            pltpu.VMEM((b_per_w,), jnp.int32),
            pltpu.VMEM((2, SEG, 2 * F), jnp.float32),
            pltpu.VMEM((2, SEG, 2 * F), jnp.float32),
            pltpu.VMEM((b_per_w,), jnp.float32),
            pltpu.VMEM((F + 16,), jnp.float32),
            pltpu.SemaphoreType.DMA,
            pltpu.SemaphoreType.DMA,
        ],
    )
    def k(x1_hbm, x2_hbm, u_hbm, v_hbm, wb_hbm, out_hbm,
          idx1_v, idx2_v, g1_v, g2_v, p1_v, p2_v, u_v, v_v, out_v, wb_v,
          sem0, sem1):
        wid = lax.axis_index("s") * info.num_cores + lax.axis_index("c")
        base = wid * b_per_w

        pltpu.sync_copy(x1_hbm.at[pl.ds(wid * n_seg, n_seg)], idx1_v)
        pltpu.sync_copy(x2_hbm.at[pl.ds(wid * n_seg, n_seg)], idx2_v)
        pltpu.sync_copy(wb_hbm, wb_v)

        for j in range(n_seg):
            for c in range(SEG // 16):
                sl = pl.ds(16 * c, 16)
                raw1 = idx1_v[j, sl]
                raw2 = idx2_v[j, sl]
                hi1 = raw1 >= HALF
                hi2 = raw2 >= HALF
                g1_v[j, sl] = raw1 - jnp.where(hi1, HALF, 0)
                g2_v[j, sl] = raw2 - jnp.where(hi2, HALF, 0)
                fl = pl.ds(j * SEG + 16 * c, 16)
                p1_v[fl] = jnp.where(hi1 & (raw1 < TAIL), 64, 0)
                p2_v[fl] = jnp.where(hi2 & (raw2 < TAIL), 64, 0)

        sems = (sem0, sem1)

        def fire(j):
            buf = j % 2
            return (
                pltpu.async_copy(u_hbm.at[g1_v.at[j]], u_v.at[buf],
                                 sems[buf]),
                pltpu.async_copy(v_hbm.at[g2_v.at[j]], v_v.at[buf],
                                 sems[buf]),
            )

        w0 = wb_v[pl.ds(0, 16)]
        w1 = wb_v[pl.ds(16, 16)]
        w2 = wb_v[pl.ds(32, 16)]
        w3 = wb_v[pl.ds(48, 16)]
        bias = wb_v[pl.ds(F, 16)]
        lane = lax.iota(jnp.int32, 16)
        perms = [jnp.bitwise_xor(lane, sh) for sh in (8, 4, 2, 1)]

        def lanesum(v):
            for p in perms:
                v = v + v.at[p].get(mode="promise_in_bounds")
            return v

        def make_group_body(j):
            buf = j % 2
            ub = u_v.at[buf]
            vb = v_v.at[buf]

            def group_body(g, carry):
                base_r = pl.multiple_of(g * 16, 16)
                pv1 = p1_v[pl.ds(j * SEG + base_r, 16)]
                pv2 = p2_v[pl.ds(j * SEG + base_r, 16)]
                acc = jnp.zeros((16,), jnp.float32)
                for r in range(16):
                    i = base_r + r
                    o1 = pl.multiple_of(pv1[r], 16)
                    o2 = pl.multiple_of(pv2[r], 16)
                    s = (ub[i, pl.ds(o1, 16)] * vb[i, pl.ds(o2, 16)]) * w0
                    s = s + (ub[i, pl.ds(o1 + 16, 16)]
                             * vb[i, pl.ds(o2 + 16, 16)]) * w1
                    s = s + (ub[i, pl.ds(o1 + 32, 16)]
                             * vb[i, pl.ds(o2 + 32, 16)]) * w2
                    s = s + (ub[i, pl.ds(o1 + 48, 16)]
                             * vb[i, pl.ds(o2 + 48, 16)]) * w3
                    acc = jnp.where(lane == r, lanesum(s), acc)
                x = acc + bias
                out_v[pl.ds(j * SEG + base_r, 16)] = 1.0 / (1.0 + jnp.exp(-x))
                return carry

            return group_body

        inflight = fire(0)
        for j in range(n_seg):
            nxt = fire(j + 1) if j + 1 < n_seg else None
            for c in inflight:
                c.wait()
            lax.fori_loop(0, SEG // 16, make_group_body(j), 0)
            inflight = nxt

        pltpu.sync_copy(out_v, out_hbm.at[pl.ds(base, b_per_w)])

    return k(x1r, x2r, u_fused, v_fused, wb)


def kernel(x1, x2, user_table, item_table, W, b):
    x1r = x1.reshape(B // SEG, SEG)
    x2r = x2.reshape(B // SEG, SEG)
    wb = jnp.concatenate([W.reshape(F), jnp.broadcast_to(b, (16,))])
    u_fused, v_fused = _fuse_tables_tc(user_table.T, item_table.T)
    out = _gmf_sc(x1r, x2r, u_fused, v_fused, wb)
    return out.reshape(B, 1)


# speedup vs baseline: 1.4347x; 1.4347x over previous
import functools

import jax
import jax.numpy as jnp
from jax import lax
from jax.experimental import pallas as pl
from jax.experimental.pallas import tpu as pltpu
from jax.experimental.pallas import tpu_sc as plsc

F = 64
B = 16384
SEG = 128
N_ROWS = 1000000
R = 4096
NB = 122
HALF = NB * R
TAIL = 2 * HALF
OUT_ROWS = HALF + (N_ROWS - TAIL)


def _fuse_tables_tc(ut, vt):
    def body(u_top, u_bot, v_top, v_bot, u_out, v_out):
        u_out[...] = jnp.concatenate([u_top[...].T, u_bot[...].T], axis=1)
        v_out[...] = jnp.concatenate([v_top[...].T, v_bot[...].T], axis=1)

    def top_map(i):
        return (0, jnp.where(i < NB, i, 2 * NB))

    def bot_map(i):
        return (0, jnp.where(i < NB, i + NB, 2 * NB))

    top_spec = pl.BlockSpec((F, R), top_map)
    bot_spec = pl.BlockSpec((F, R), bot_map)
    out_spec = pl.BlockSpec((R, 2 * F), lambda i: (i, 0))
    out_shape = jax.ShapeDtypeStruct((OUT_ROWS, 2 * F), jnp.float32)
    return pl.pallas_call(
        body,
        grid=(NB + 1,),
        in_specs=[top_spec, bot_spec, top_spec, bot_spec],
        out_specs=[out_spec, out_spec],
        out_shape=[out_shape, out_shape],
    )(ut, ut, vt, vt)


def _gmf_sc(x1r, x2r, u_fused, v_fused, wb):
    info = plsc.get_sparse_core_info()
    nw = info.num_cores * info.num_subcores
    b_per_w = B // nw
    n_seg = b_per_w // SEG

    mesh = plsc.VectorSubcoreMesh(core_axis_name="c", subcore_axis_name="s")

    @functools.partial(
        pl.kernel,
        mesh=mesh,
        out_type=jax.ShapeDtypeStruct((B,), jnp.float32),
        scratch_types=[
            pltpu.VMEM((n_seg, SEG), jnp.int32),
            pltpu.VMEM((n_seg, SEG), jnp.int32),
            pltpu.VMEM((n_seg, SEG), jnp.int32),
            pltpu.VMEM((n_seg, SEG), jnp.int32),
            pltpu.VMEM((b_per_w,), jnp.int32),
            pltpu.VMEM((b_per_w,), jnp.int32),
            pltpu.VMEM((2, SEG, 2 * F), jnp.float32),
            pltpu.VMEM((2, SEG, 2 * F), jnp.float32),
            pltpu.VMEM((b_per_w,), jnp.float32),
            pltpu.VMEM((F + 16,), jnp.float32),
            pltpu.SemaphoreType.DMA,
            pltpu.SemaphoreType.DMA,
        ],
    )
    def k(x1_hbm, x2_hbm, u_hbm, v_hbm, wb_hbm, out_hbm,
          idx1_v, idx2_v, g1_v, g2_v, p1_v, p2_v, u_v, v_v, out_v, wb_v,
          sem0, sem1):
        wid = lax.axis_index("s") * info.num_cores + lax.axis_index("c")
        base = wid * b_per_w

        pltpu.sync_copy(x1_hbm.at[pl.ds(wid * n_seg, n_seg)], idx1_v)
        pltpu.sync_copy(x2_hbm.at[pl.ds(wid * n_seg, n_seg)], idx2_v)
        pltpu.sync_copy(wb_hbm, wb_v)

        for j in range(n_seg):
            for c in range(SEG // 16):
                sl = pl.ds(16 * c, 16)
                raw1 = idx1_v[j, sl]
                raw2 = idx2_v[j, sl]
                hi1 = raw1 >= HALF
                hi2 = raw2 >= HALF
                g1_v[j, sl] = raw1 - jnp.where(hi1, HALF, 0)
                g2_v[j, sl] = raw2 - jnp.where(hi2, HALF, 0)
                fl = pl.ds(j * SEG + 16 * c, 16)
                p1_v[fl] = jnp.where(hi1 & (raw1 < TAIL), 64, 0)
                p2_v[fl] = jnp.where(hi2 & (raw2 < TAIL), 64, 0)

        sems = (sem0, sem1)

        def fire(j):
            buf = j % 2
            return (
                pltpu.async_copy(u_hbm.at[g1_v.at[j]], u_v.at[buf],
                                 sems[buf]),
                pltpu.async_copy(v_hbm.at[g2_v.at[j]], v_v.at[buf],
                                 sems[buf]),
            )

        w0 = wb_v[pl.ds(0, 16)]
        w1 = wb_v[pl.ds(16, 16)]
        w2 = wb_v[pl.ds(32, 16)]
        w3 = wb_v[pl.ds(48, 16)]
        bias = wb_v[pl.ds(F, 16)]
        lane = lax.iota(jnp.int32, 16)
        perms = [jnp.bitwise_xor(lane, sh) for sh in (8, 4, 2, 1)]

        def lanesum(v):
            for p in perms:
                v = v + v.at[p].get(mode="promise_in_bounds")
            return v

        def make_group_body(j):
            buf = j % 2
            ub = u_v.at[buf]
            vb = v_v.at[buf]

            def group_body(g, carry):
                base_r = pl.multiple_of(g * 16, 16)
                pv1 = p1_v[pl.ds(j * SEG + base_r, 16)]
                pv2 = p2_v[pl.ds(j * SEG + base_r, 16)]
                acc = jnp.zeros((16,), jnp.float32)
                for r in range(16):
                    i = base_r + r
                    o1 = pl.multiple_of(pv1[r], 16)
                    o2 = pl.multiple_of(pv2[r], 16)
                    s = (ub[i, pl.ds(o1, 16)] * vb[i, pl.ds(o2, 16)]) * w0
                    s = s + (ub[i, pl.ds(o1 + 16, 16)]
                             * vb[i, pl.ds(o2 + 16, 16)]) * w1
                    s = s + (ub[i, pl.ds(o1 + 32, 16)]
                             * vb[i, pl.ds(o2 + 32, 16)]) * w2
                    s = s + (ub[i, pl.ds(o1 + 48, 16)]
                             * vb[i, pl.ds(o2 + 48, 16)]) * w3
                    acc = jnp.where(lane == r, lanesum(s), acc)
                x = acc + bias
                out_v[pl.ds(j * SEG + base_r, 16)] = 1.0 / (1.0 + jnp.exp(-x))
                return carry

            return group_body

        inflight = fire(0)
        for j in range(n_seg):
            nxt = fire(j + 1) if j + 1 < n_seg else None
            for c in inflight:
                c.wait()
            lax.fori_loop(0, SEG // 16, make_group_body(j), 0)
            inflight = nxt

        pltpu.sync_copy(out_v, out_hbm.at[pl.ds(base, b_per_w)])

    return k(x1r, x2r, u_fused, v_fused, wb)


def kernel(x1, x2, user_table, item_table, W, b):
    x1r = x1.reshape(B // SEG, SEG)
    x2r = x2.reshape(B // SEG, SEG)
    wb = jnp.concatenate([W.reshape(F), jnp.broadcast_to(b, (16,))])
    u_fused, v_fused = _fuse_tables_tc(user_table.T, item_table.T)
    out = _gmf_sc(x1r, x2r, u_fused, v_fused, wb)
    return out.reshape(B, 1)


# speedup vs baseline: 1.5272x; 1.0644x over previous
import functools

import jax
import jax.numpy as jnp
from jax import lax
from jax.experimental import pallas as pl
from jax.experimental.pallas import tpu as pltpu
from jax.experimental.pallas import tpu_sc as plsc

F = 64
B = 16384
SEG = 128
N_ROWS = 1000000
R = 8192
NB = 61
HALF = NB * R
TAIL = 2 * HALF
OUT_ROWS = HALF + (N_ROWS - TAIL)


def _fuse_tables_tc(ut, vt):
    def body(u_top, u_bot, v_top, v_bot, u_out, v_out):
        u_out[...] = jnp.concatenate([u_top[...].T, u_bot[...].T], axis=1)
        v_out[...] = jnp.concatenate([v_top[...].T, v_bot[...].T], axis=1)

    def top_map(i):
        return (0, jnp.where(i < NB, i, 2 * NB))

    def bot_map(i):
        return (0, jnp.where(i < NB, i + NB, 2 * NB))

    top_spec = pl.BlockSpec((F, R), top_map)
    bot_spec = pl.BlockSpec((F, R), bot_map)
    out_spec = pl.BlockSpec((R, 2 * F), lambda i: (i, 0))
    out_shape = jax.ShapeDtypeStruct((OUT_ROWS, 2 * F), jnp.float32)
    return pl.pallas_call(
        body,
        grid=(NB + 1,),
        in_specs=[top_spec, bot_spec, top_spec, bot_spec],
        out_specs=[out_spec, out_spec],
        out_shape=[out_shape, out_shape],
    )(ut, ut, vt, vt)


def _gmf_sc(x1r, x2r, u_fused, v_fused, wb):
    info = plsc.get_sparse_core_info()
    nw = info.num_cores * info.num_subcores
    b_per_w = B // nw
    n_seg = b_per_w // SEG

    mesh = plsc.VectorSubcoreMesh(core_axis_name="c", subcore_axis_name="s")

    @functools.partial(
        pl.kernel,
        mesh=mesh,
        out_type=jax.ShapeDtypeStruct((B,), jnp.float32),
        scratch_types=[
            pltpu.VMEM((n_seg, SEG), jnp.int32),
            pltpu.VMEM((n_seg, SEG), jnp.int32),
            pltpu.VMEM((n_seg, SEG), jnp.int32),
            pltpu.VMEM((n_seg, SEG), jnp.int32),
            pltpu.VMEM((b_per_w,), jnp.int32),
            pltpu.VMEM((b_per_w,), jnp.int32),
            pltpu.VMEM((2, SEG, 2 * F), jnp.float32),
            pltpu.VMEM((2, SEG, 2 * F), jnp.float32),
            pltpu.VMEM((b_per_w,), jnp.float32),
            pltpu.VMEM((F + 16,), jnp.float32),
            pltpu.SemaphoreType.DMA,
            pltpu.SemaphoreType.DMA,
        ],
    )
    def k(x1_hbm, x2_hbm, u_hbm, v_hbm, wb_hbm, out_hbm,
          idx1_v, idx2_v, g1_v, g2_v, p1_v, p2_v, u_v, v_v, out_v, wb_v,
          sem0, sem1):
        wid = lax.axis_index("s") * info.num_cores + lax.axis_index("c")
        base = wid * b_per_w

        pltpu.sync_copy(x1_hbm.at[pl.ds(wid * n_seg, n_seg)], idx1_v)
        pltpu.sync_copy(x2_hbm.at[pl.ds(wid * n_seg, n_seg)], idx2_v)
        pltpu.sync_copy(wb_hbm, wb_v)

        for j in range(n_seg):
            for c in range(SEG // 16):
                sl = pl.ds(16 * c, 16)
                raw1 = idx1_v[j, sl]
                raw2 = idx2_v[j, sl]
                hi1 = raw1 >= HALF
                hi2 = raw2 >= HALF
                g1_v[j, sl] = raw1 - jnp.where(hi1, HALF, 0)
                g2_v[j, sl] = raw2 - jnp.where(hi2, HALF, 0)
                fl = pl.ds(j * SEG + 16 * c, 16)
                p1_v[fl] = jnp.where(hi1 & (raw1 < TAIL), 64, 0)
                p2_v[fl] = jnp.where(hi2 & (raw2 < TAIL), 64, 0)

        sems = (sem0, sem1)

        def fire(j):
            buf = j % 2
            return (
                pltpu.async_copy(u_hbm.at[g1_v.at[j]], u_v.at[buf],
                                 sems[buf]),
                pltpu.async_copy(v_hbm.at[g2_v.at[j]], v_v.at[buf],
                                 sems[buf]),
            )

        w0 = wb_v[pl.ds(0, 16)]
        w1 = wb_v[pl.ds(16, 16)]
        w2 = wb_v[pl.ds(32, 16)]
        w3 = wb_v[pl.ds(48, 16)]
        bias = wb_v[pl.ds(F, 16)]
        lane = lax.iota(jnp.int32, 16)
        perms = [jnp.bitwise_xor(lane, sh) for sh in (8, 4, 2, 1)]

        def lanesum(v):
            for p in perms:
                v = v + v.at[p].get(mode="promise_in_bounds")
            return v

        def make_group_body(j):
            buf = j % 2
            ub = u_v.at[buf]
            vb = v_v.at[buf]

            def group_body(g, carry):
                base_r = pl.multiple_of(g * 16, 16)
                pv1 = p1_v[pl.ds(j * SEG + base_r, 16)]
                pv2 = p2_v[pl.ds(j * SEG + base_r, 16)]
                acc = jnp.zeros((16,), jnp.float32)
                for r in range(16):
                    i = base_r + r
                    o1 = pl.multiple_of(pv1[r], 16)
                    o2 = pl.multiple_of(pv2[r], 16)
                    s = (ub[i, pl.ds(o1, 16)] * vb[i, pl.ds(o2, 16)]) * w0
                    s = s + (ub[i, pl.ds(o1 + 16, 16)]
                             * vb[i, pl.ds(o2 + 16, 16)]) * w1
                    s = s + (ub[i, pl.ds(o1 + 32, 16)]
                             * vb[i, pl.ds(o2 + 32, 16)]) * w2
                    s = s + (ub[i, pl.ds(o1 + 48, 16)]
                             * vb[i, pl.ds(o2 + 48, 16)]) * w3
                    acc = jnp.where(lane == r, lanesum(s), acc)
                x = acc + bias
                out_v[pl.ds(j * SEG + base_r, 16)] = 1.0 / (1.0 + jnp.exp(-x))
                return carry

            return group_body

        inflight = fire(0)
        for j in range(n_seg):
            nxt = fire(j + 1) if j + 1 < n_seg else None
            for c in inflight:
                c.wait()
            lax.fori_loop(0, SEG // 16, make_group_body(j), 0)
            inflight = nxt

        pltpu.sync_copy(out_v, out_hbm.at[pl.ds(base, b_per_w)])

    return k(x1r, x2r, u_fused, v_fused, wb)


def kernel(x1, x2, user_table, item_table, W, b):
    x1r = x1.reshape(B // SEG, SEG)
    x2r = x2.reshape(B // SEG, SEG)
    wb = jnp.concatenate([W.reshape(F), jnp.broadcast_to(b, (16,))])
    u_fused, v_fused = _fuse_tables_tc(user_table.T, item_table.T)
    out = _gmf_sc(x1r, x2r, u_fused, v_fused, wb)
    return out.reshape(B, 1)
